# Initial kernel scaffold; baseline (speedup 1.0000x reference)
import functools

import jax
import jax.numpy as jnp
from jax import lax
from jax.experimental import pallas as pl
from jax.experimental.pallas import tpu as pltpu
from jax.experimental.pallas import tpu_sc as plsc

N = 10000
D = 128
H = 2
FF = 256
NP = 10112
SPAN = NP // 16
C = 128

_mesh = plsc.VectorSubcoreMesh(core_axis_name="c", subcore_axis_name="s")


def _head_body(x_ref, wt_ref, a_ref, xt_ref, atab_ref, m_ref):
    x = x_ref[...]
    xt = jnp.dot(x, wt_ref[...], preferred_element_type=jnp.float32)
    xt_ref[:NP] = xt[:, :D]
    xt_ref[NP:] = xt[:, D:]
    atab = jnp.dot(xt, a_ref[...], preferred_element_type=jnp.float32)
    atab_ref[...] = atab
    mx = jnp.max(atab, axis=0, keepdims=True)
    msum = mx[:, 0:2] + mx[:, 2:4]
    msum = jnp.where(msum >= 0.0, msum, msum * 0.2)
    m_ref[...] = jnp.concatenate(
        [jnp.broadcast_to(msum[:, 0:1], (1, 16)),
         jnp.broadcast_to(msum[:, 1:2], (1, 16))], axis=0)


def _run_head(xpad, wt, amat):
    return pl.pallas_call(
        _head_body,
        out_shape=(
            jax.ShapeDtypeStruct((2 * NP, D), jnp.float32),
            jax.ShapeDtypeStruct((NP, 16), jnp.float32),
            jax.ShapeDtypeStruct((2, 16), jnp.float32),
        ),
    )(xpad, wt, amat)


def _make_edge_logits(epp, nch):
    epw = epp // 32

    @functools.partial(
        pl.kernel,
        out_type=(
            jax.ShapeDtypeStruct((2 * epp,), jnp.float32),
            jax.ShapeDtypeStruct((4 * NP,), jnp.float32),
        ),
        mesh=_mesh,
        scratch_types=[
            pltpu.VMEM((C,), jnp.int32),
            pltpu.VMEM((C,), jnp.int32),
            pltpu.VMEM((C, 16), jnp.float32),
            pltpu.VMEM((C, 16), jnp.float32),
            pltpu.VMEM((C,), jnp.float32),
            pltpu.VMEM((C,), jnp.float32),
            pltpu.VMEM((2, 16), jnp.float32),
            pltpu.VMEM_SHARED((NP,), jnp.float32),
            pltpu.VMEM_SHARED((NP,), jnp.float32),
            pltpu.SemaphoreType.DMA,
        ],
    )
    def edge_logits(src_hbm, dst_hbm, atab_hbm, m_hbm, znp_hbm,
                    w_out, pden_out,
                    srcv, dstv, asrc, adst, w0, w1, mv, den0, den1, sem):
        c = lax.axis_index("c")
        s = lax.axis_index("s")
        wid = s * 2 + c
        sl = pl.ds(s * SPAN, SPAN)
        pltpu.sync_copy(m_hbm, mv)
        pltpu.sync_copy(znp_hbm.at[sl], den0.at[sl])
        pltpu.sync_copy(znp_hbm.at[sl], den1.at[sl])
        plsc.subcore_barrier()
        iota = lax.iota(jnp.int32, 16)

        def chunk(i, carry):
            base = wid * epw + i * C
            pltpu.sync_copy(src_hbm.at[pl.ds(base, C)], srcv)
            pltpu.sync_copy(dst_hbm.at[pl.ds(base, C)], dstv)
            pltpu.async_copy(atab_hbm.at[srcv], asrc, sem).wait()
            pltpu.async_copy(atab_hbm.at[dstv], adst, sem).wait()
            for k in range(C // 16):
                e16 = iota + (k * 16)
                for h, wbuf in ((0, w0), (1, w1)):
                    a_s = plsc.load_gather(
                        asrc, [e16, jnp.full((16,), h, jnp.int32)])
                    a_d = plsc.load_gather(
                        adst, [e16, jnp.full((16,), 2 + h, jnp.int32)])
                    al = a_s + a_d
                    al = jnp.where(al >= 0.0, al, al * 0.2)
                    wbuf[pl.ds(k * 16, 16)] = jnp.exp(al - mv[h])
            pltpu.sync_copy(w0, w_out.at[pl.ds(base, C)])
            pltpu.sync_copy(w1, w_out.at[pl.ds(epp + base, C)])
            pltpu.sync_copy(w0, den0.at[dstv], add=True)
            pltpu.sync_copy(w1, den1.at[dstv], add=True)
            return carry

        lax.fori_loop(0, nch, chunk, 0)
        plsc.subcore_barrier()
        pltpu.sync_copy(den0.at[sl],
                        pden_out.at[pl.ds(c * 2 * NP + s * SPAN, SPAN)])
        pltpu.sync_copy(den1.at[sl],
                        pden_out.at[pl.ds(c * 2 * NP + NP + s * SPAN, SPAN)])

    return edge_logits


def _make_aggregate(epp, nch):
    epw = epp // 16

    @functools.partial(
        pl.kernel,
        out_type=jax.ShapeDtypeStruct((2 * NP, D), jnp.float32),
        mesh=_mesh,
        scratch_types=[
            pltpu.VMEM((C,), jnp.int32),
            pltpu.VMEM((C,), jnp.int32),
            pltpu.VMEM((C,), jnp.float32),
            pltpu.VMEM((NP,), jnp.float32),
            pltpu.VMEM((NP,), jnp.float32),
            pltpu.VMEM((C, D), jnp.float32),
            pltpu.VMEM_SHARED((NP, D), jnp.float32),
            pltpu.SemaphoreType.DMA,
        ],
    )
    def aggregate(src_hbm, dst_hbm, w_hbm, pden_hbm, xt_hbm, z2d_hbm,
                  agg_out,
                  srcv, dstv, wv, denv, tmpv, rows, agg_sp, sem):
        c = lax.axis_index("c")
        s = lax.axis_index("s")
        sl = pl.ds(s * SPAN, SPAN)
        pltpu.sync_copy(pden_hbm.at[pl.ds(c * NP, NP)], denv)
        pltpu.sync_copy(pden_hbm.at[pl.ds(2 * NP + c * NP, NP)], tmpv)

        def addj(j, carry):
            q = pl.ds(j * 16, 16)
            denv[q] = denv[q] + tmpv[q]
            return carry

        lax.fori_loop(0, NP // 16, addj, 0)
        pltpu.sync_copy(z2d_hbm, agg_sp.at[sl])
        plsc.subcore_barrier()
        iota = lax.iota(jnp.int32, 16)

        def chunk(i, carry):
            base = s * epw + i * C
            pltpu.sync_copy(src_hbm.at[pl.ds(base, C)], srcv)
            pltpu.sync_copy(dst_hbm.at[pl.ds(base, C)], dstv)
            pltpu.sync_copy(w_hbm.at[pl.ds(c * epp + base, C)], wv)

            def shiftj(j, cr):
                q = pl.ds(j * 16, 16)
                srcv[q] = srcv[q] + c * NP
                return cr

            lax.fori_loop(0, C // 16, shiftj, 0)
            pltpu.async_copy(xt_hbm.at[srcv], rows, sem).wait()

            def kbody(k, cr):
                q = pl.ds(k * 16, 16)
                e16 = iota + k * 16
                den16 = plsc.load_gather(denv, [dstv[q]])
                cf = wv[q] / (den16 + 1e-16)
                for d in range(D):
                    dd = jnp.full((16,), d, jnp.int32)
                    v = plsc.load_gather(rows, [e16, dd])
                    plsc.store_scatter(rows, [e16, dd], v * cf)
                return cr

            lax.fori_loop(0, C // 16, kbody, 0)
            pltpu.sync_copy(rows, agg_sp.at[dstv], add=True)
            return carry

        lax.fori_loop(0, nch, chunk, 0)
        plsc.subcore_barrier()
        pltpu.sync_copy(agg_sp.at[sl],
                        agg_out.at[pl.ds(c * NP + s * SPAN, SPAN)])

    return aggregate


def _tail_body(agg_ref, na_ref, bg_ref, wl_ref, l1w_ref, l1b_ref,
               w1_ref, b1_ref, w2_ref, b2_ref, l2w_ref, l2b_ref, out_ref):
    whole = agg_ref[...]
    gat = jnp.concatenate([whole[:N], whole[NP:NP + N]], axis=1) + bg_ref[...]
    x1 = jnp.dot(gat, wl_ref[...], preferred_element_type=jnp.float32)
    x1 = x1 + na_ref[...]
    mu = jnp.mean(x1)
    ce = x1 - mu
    var = jnp.mean(ce * ce)
    x = ce * lax.rsqrt(var + 1e-5) * l1w_ref[...] + l1b_ref[...]
    h = jnp.maximum(
        jnp.dot(x, w1_ref[...], preferred_element_type=jnp.float32)
        + b1_ref[...], 0.0)
    h2 = jnp.dot(h, w2_ref[...], preferred_element_type=jnp.float32)
    y = x + h2 + b2_ref[...]
    mu2 = jnp.mean(y)
    ce2 = y - mu2
    var2 = jnp.mean(ce2 * ce2)
    out_ref[...] = ce2 * lax.rsqrt(var2 + 1e-5) * l2w_ref[...] + l2b_ref[...]


def _run_tail(agg, node_attr, bg, wl, l1w, l1b, w1, b1, w2, b2, l2w, l2b):
    return pl.pallas_call(
        _tail_body,
        out_shape=jax.ShapeDtypeStruct((N, D), jnp.float32),
    )(agg, node_attr, bg, wl, l1w, l1b, w1, b1, w2, b2, l2w, l2b)


def kernel(node_attr, edge_index, W_gat, att_src, att_dst, b_gat, W_lin,
           ln1_w, ln1_b, W_ff1, b_ff1, W_ff2, b_ff2, ln2_w, ln2_b):
    e = edge_index.shape[1]
    ep = e + N
    epp = -(-ep // (32 * C)) * (32 * C)
    nch_a = epp // (32 * C)
    nch_b = epp // (16 * C)

    f32 = jnp.float32
    xpad = jnp.zeros((NP, D), f32).at[:N].set(node_attr)
    loop = jnp.arange(N, dtype=jnp.int32)
    padi = jnp.full((epp - ep,), N, jnp.int32)
    src = jnp.concatenate([edge_index[0].astype(jnp.int32), loop, padi])
    dst = jnp.concatenate([edge_index[1].astype(jnp.int32), loop, padi])

    wt = W_gat.T
    amat = jnp.zeros((2 * D, 16), f32)
    amat = amat.at[:D, 0].set(att_src[0, 0]).at[D:, 1].set(att_src[0, 1])
    amat = amat.at[:D, 2].set(att_dst[0, 0]).at[D:, 3].set(att_dst[0, 1])

    xt2, atab, m2 = _run_head(xpad, wt, amat)

    w_flat, pden = _make_edge_logits(epp, nch_a)(
        src, dst, atab, m2, jnp.zeros((NP,), f32))

    agg = _make_aggregate(epp, nch_b)(
        src, dst, w_flat, pden, xt2, jnp.zeros((SPAN, D), f32))

    return _run_tail(
        agg, node_attr, b_gat.reshape(1, 2 * D), W_lin.T,
        ln1_w.reshape(1, D), ln1_b.reshape(1, D),
        W_ff1.T, b_ff1.reshape(1, FF), W_ff2.T, b_ff2.reshape(1, D),
        ln2_w.reshape(1, D), ln2_b.reshape(1, D))


# speedup vs baseline: 18.4682x; 18.4682x over previous
import functools

import jax
import jax.numpy as jnp
from jax import lax
from jax.experimental import pallas as pl
from jax.experimental.pallas import tpu as pltpu
from jax.experimental.pallas import tpu_sc as plsc

N = 10000
D = 128
H = 2
FF = 256
NP = 10112
SPAN = NP // 16
C = 128

@functools.cache
def _mesh():
    return plsc.VectorSubcoreMesh(core_axis_name="c", subcore_axis_name="s")


def _head_body(x_ref, wt_ref, a_ref, xt_ref, atab_ref, m_ref):
    x = x_ref[...]
    xt = jnp.dot(x, wt_ref[...], preferred_element_type=jnp.float32)
    xt_ref[:NP] = xt[:, :D]
    xt_ref[NP:] = xt[:, D:]
    atab = jnp.dot(xt, a_ref[...], preferred_element_type=jnp.float32)
    atab_ref[...] = atab
    mx = jnp.max(atab, axis=0, keepdims=True)
    msum = mx[:, 0:2] + mx[:, 2:4]
    msum = jnp.where(msum >= 0.0, msum, msum * 0.2)
    m_ref[...] = jnp.concatenate(
        [jnp.broadcast_to(msum[:, 0:1], (1, 16)),
         jnp.broadcast_to(msum[:, 1:2], (1, 16))], axis=0)


def _run_head(xpad, wt, amat):
    return pl.pallas_call(
        _head_body,
        out_shape=(
            jax.ShapeDtypeStruct((2 * NP, D), jnp.float32),
            jax.ShapeDtypeStruct((NP, 16), jnp.float32),
            jax.ShapeDtypeStruct((2, 16), jnp.float32),
        ),
    )(xpad, wt, amat)


def _make_edge_logits(epp, nch):
    epw = epp // 32

    @functools.partial(
        pl.kernel,
        out_type=(
            jax.ShapeDtypeStruct((2 * epp,), jnp.float32),
            jax.ShapeDtypeStruct((4 * NP,), jnp.float32),
        ),
        mesh=_mesh(),
        scratch_types=[
            pltpu.VMEM((C,), jnp.int32),
            pltpu.VMEM((C,), jnp.int32),
            pltpu.VMEM((C,), jnp.float32),
            pltpu.VMEM((C,), jnp.float32),
            pltpu.VMEM((C,), jnp.float32),
            pltpu.VMEM((C,), jnp.float32),
            pltpu.VMEM((C,), jnp.float32),
            pltpu.VMEM((C,), jnp.float32),
            pltpu.VMEM((2, 16), jnp.float32),
            pltpu.VMEM_SHARED((NP,), jnp.float32),
            pltpu.VMEM_SHARED((NP,), jnp.float32),
            pltpu.SemaphoreType.DMA,
        ],
    )
    def edge_logits(src_hbm, dst_hbm, as0_hbm, as1_hbm, ad0_hbm, ad1_hbm,
                    m_hbm,
                    w_out, pden_out,
                    srcv, dstv, s0v, s1v, d0v, d1v, w0, w1, mv,
                    den0, den1, sem):
        c = lax.axis_index("c")
        s = lax.axis_index("s")
        wid = s * 2 + c
        base_n = s * SPAN
        pltpu.sync_copy(m_hbm, mv)
        for k in range(C // 16):
            w0[pl.ds(k * 16, 16)] = jnp.zeros((16,), jnp.float32)
        for off, ln in ((0, C), (C, C), (2 * C, C), (3 * C, C), (4 * C, 120)):
            pltpu.sync_copy(w0.at[pl.ds(0, ln)],
                            den0.at[pl.ds(base_n + off, ln)])
            pltpu.sync_copy(w0.at[pl.ds(0, ln)],
                            den1.at[pl.ds(base_n + off, ln)])
        plsc.subcore_barrier()
        m0 = mv[0]
        m1 = mv[1]

        def chunk(i, carry):
            base = wid * epw + i * C
            pltpu.sync_copy(src_hbm.at[pl.ds(base, C)], srcv)
            pltpu.sync_copy(dst_hbm.at[pl.ds(base, C)], dstv)
            pltpu.async_copy(as0_hbm.at[srcv], s0v, sem).wait()
            pltpu.async_copy(as1_hbm.at[srcv], s1v, sem).wait()
            pltpu.async_copy(ad0_hbm.at[dstv], d0v, sem).wait()
            pltpu.async_copy(ad1_hbm.at[dstv], d1v, sem).wait()
            for k in range(C // 16):
                q = pl.ds(k * 16, 16)
                al0 = s0v[q] + d0v[q]
                al0 = jnp.where(al0 >= 0.0, al0, al0 * 0.2)
                w0[q] = jnp.exp(al0 - m0)
                al1 = s1v[q] + d1v[q]
                al1 = jnp.where(al1 >= 0.0, al1, al1 * 0.2)
                w1[q] = jnp.exp(al1 - m1)
            pltpu.sync_copy(w0, w_out.at[pl.ds(base, C)])
            pltpu.sync_copy(w1, w_out.at[pl.ds(epp + base, C)])
            pltpu.sync_copy(w0, den0.at[dstv], add=True)
            pltpu.sync_copy(w1, den1.at[dstv], add=True)
            return carry

        lax.fori_loop(0, nch, chunk, 0)
        plsc.subcore_barrier()
        for off, ln in ((0, C), (C, C), (2 * C, C), (3 * C, C), (4 * C, 120)):
            pltpu.sync_copy(den0.at[pl.ds(base_n + off, ln)],
                            w0.at[pl.ds(0, ln)])
            pltpu.sync_copy(
                w0.at[pl.ds(0, ln)],
                pden_out.at[pl.ds(c * 2 * NP + base_n + off, ln)])
            pltpu.sync_copy(den1.at[pl.ds(base_n + off, ln)],
                            w1.at[pl.ds(0, ln)])
            pltpu.sync_copy(
                w1.at[pl.ds(0, ln)],
                pden_out.at[pl.ds(c * 2 * NP + NP + base_n + off, ln)])

    return edge_logits


def _make_aggregate(epp, nch):
    epw = epp // 16

    @functools.partial(
        pl.kernel,
        out_type=jax.ShapeDtypeStruct((2 * NP, D), jnp.float32),
        mesh=_mesh(),
        scratch_types=[
            pltpu.VMEM((C,), jnp.int32),
            pltpu.VMEM((C,), jnp.int32),
            pltpu.VMEM((C,), jnp.int32),
            pltpu.VMEM((C,), jnp.float32),
            pltpu.VMEM((C,), jnp.float32),
            pltpu.VMEM((C,), jnp.float32),
            pltpu.VMEM((C, D), jnp.float32),
            pltpu.VMEM_SHARED((NP, D), jnp.float32),
            pltpu.SemaphoreType.DMA,
        ],
    )
    def aggregate(src_hbm, dst_hbm, w_hbm, pden_hbm, xt_hbm,
                  agg_out,
                  srcv, dstv, idxv, wv, g1v, g2v, rows, agg_sp, sem):
        c = lax.axis_index("c")
        s = lax.axis_index("s")
        base_n = s * SPAN

        def zr(j, cr):
            for r in range(D // 16):
                rows[j, pl.ds(r * 16, 16)] = jnp.zeros((16,), jnp.float32)
            return cr

        lax.fori_loop(0, C, zr, 0)
        for off, ln in ((0, C), (C, C), (2 * C, C), (3 * C, C), (4 * C, 120)):
            pltpu.sync_copy(rows.at[pl.ds(0, ln)],
                            agg_sp.at[pl.ds(base_n + off, ln)])
        plsc.subcore_barrier()

        def chunk(i, carry):
            base = s * epw + i * C
            pltpu.sync_copy(src_hbm.at[pl.ds(base, C)], srcv)
            pltpu.sync_copy(dst_hbm.at[pl.ds(base, C)], dstv)
            pltpu.sync_copy(w_hbm.at[pl.ds(c * epp + base, C)], wv)

            off1 = c * NP
            off2 = 2 * NP + c * NP
            for k in range(C // 16):
                q = pl.ds(k * 16, 16)
                idxv[q] = dstv[q] + off1
            pltpu.async_copy(pden_hbm.at[idxv], g1v, sem).wait()
            for k in range(C // 16):
                q = pl.ds(k * 16, 16)
                idxv[q] = dstv[q] + off2
            pltpu.async_copy(pden_hbm.at[idxv], g2v, sem).wait()

            for k in range(C // 16):
                q = pl.ds(k * 16, 16)
                idxv[q] = srcv[q] + c * NP
            pltpu.async_copy(xt_hbm.at[idxv], rows, sem).wait()

            for k in range(C // 16):
                q = pl.ds(k * 16, 16)
                cf = wv[q] / (g1v[q] + g2v[q] + 1e-16)
                for u in range(16):
                    e = k * 16 + u
                    cfu = cf.at[jnp.full((16,), u, jnp.int32)].get(
                        mode="promise_in_bounds")
                    for r in range(D // 16):
                        q2 = pl.ds(r * 16, 16)
                        rows[e, q2] = rows[e, q2] * cfu
            pltpu.sync_copy(rows, agg_sp.at[dstv], add=True)
            return carry

        lax.fori_loop(0, nch, chunk, 0)
        plsc.subcore_barrier()
        for off, ln in ((0, C), (C, C), (2 * C, C), (3 * C, C), (4 * C, 120)):
            pltpu.sync_copy(agg_sp.at[pl.ds(base_n + off, ln)],
                            rows.at[pl.ds(0, ln)])
            pltpu.sync_copy(rows.at[pl.ds(0, ln)],
                            agg_out.at[pl.ds(c * NP + base_n + off, ln)])

    return aggregate


def _tail_body(agg_ref, na_ref, bg_ref, wl_ref, l1w_ref, l1b_ref,
               w1_ref, b1_ref, w2_ref, b2_ref, l2w_ref, l2b_ref, out_ref):
    whole = agg_ref[...]
    gat = jnp.concatenate([whole[:N], whole[NP:NP + N]], axis=1) + bg_ref[...]
    x1 = jnp.dot(gat, wl_ref[...], preferred_element_type=jnp.float32)
    x1 = x1 + na_ref[...]
    mu = jnp.mean(x1)
    ce = x1 - mu
    var = jnp.mean(ce * ce)
    x = ce * lax.rsqrt(var + 1e-5) * l1w_ref[...] + l1b_ref[...]
    h = jnp.maximum(
        jnp.dot(x, w1_ref[...], preferred_element_type=jnp.float32)
        + b1_ref[...], 0.0)
    h2 = jnp.dot(h, w2_ref[...], preferred_element_type=jnp.float32)
    y = x + h2 + b2_ref[...]
    mu2 = jnp.mean(y)
    ce2 = y - mu2
    var2 = jnp.mean(ce2 * ce2)
    out_ref[...] = ce2 * lax.rsqrt(var2 + 1e-5) * l2w_ref[...] + l2b_ref[...]


def _run_tail(agg, node_attr, bg, wl, l1w, l1b, w1, b1, w2, b2, l2w, l2b):
    return pl.pallas_call(
        _tail_body,
        out_shape=jax.ShapeDtypeStruct((N, D), jnp.float32),
    )(agg, node_attr, bg, wl, l1w, l1b, w1, b1, w2, b2, l2w, l2b)


def kernel(node_attr, edge_index, W_gat, att_src, att_dst, b_gat, W_lin,
           ln1_w, ln1_b, W_ff1, b_ff1, W_ff2, b_ff2, ln2_w, ln2_b):
    e = edge_index.shape[1]
    ep = e + N
    epp = -(-ep // (32 * C)) * (32 * C)
    nch_a = epp // (32 * C)
    nch_b = epp // (16 * C)

    f32 = jnp.float32
    xpad = jnp.zeros((NP, D), f32).at[:N].set(node_attr)
    loop = jnp.arange(N, dtype=jnp.int32)
    padi = jnp.full((epp - ep,), N, jnp.int32)
    src = jnp.concatenate([edge_index[0].astype(jnp.int32), loop, padi])
    dst = jnp.concatenate([edge_index[1].astype(jnp.int32), loop, padi])

    wt = W_gat.T
    amat = jnp.zeros((2 * D, 16), f32)
    amat = amat.at[:D, 0].set(att_src[0, 0]).at[D:, 1].set(att_src[0, 1])
    amat = amat.at[:D, 2].set(att_dst[0, 0]).at[D:, 3].set(att_dst[0, 1])

    xt2, atab, m2 = _run_head(xpad, wt, amat)

    w_flat, pden = _make_edge_logits(epp, nch_a)(
        src, dst,
        atab[:, 0] + 0.0, atab[:, 1] + 0.0,
        atab[:, 2] + 0.0, atab[:, 3] + 0.0,
        m2)

    agg = _make_aggregate(epp, nch_b)(
        src, dst, w_flat, pden, xt2)

    return _run_tail(
        agg, node_attr, b_gat.reshape(1, 2 * D), W_lin.T,
        ln1_w.reshape(1, D), ln1_b.reshape(1, D),
        W_ff1.T, b_ff1.reshape(1, FF), W_ff2.T, b_ff2.reshape(1, D),
        ln2_w.reshape(1, D), ln2_b.reshape(1, D))


# speedup vs baseline: 31.5738x; 1.7096x over previous
import functools

import jax
import jax.numpy as jnp
from jax import lax
from jax.experimental import pallas as pl
from jax.experimental.pallas import tpu as pltpu
from jax.experimental.pallas import tpu_sc as plsc

N = 10000
D = 128
H = 2
FF = 256
NP = 10112
SPAN = NP // 16
C = 128

@functools.cache
def _mesh():
    return plsc.VectorSubcoreMesh(core_axis_name="c", subcore_axis_name="s")


def _head_body(x_ref, wt_ref, a_ref, xt_ref, atab_ref, m_ref):
    x = x_ref[...]
    xt = jnp.dot(x, wt_ref[...], preferred_element_type=jnp.float32)
    xt_ref[:NP] = xt[:, :D]
    xt_ref[NP:] = xt[:, D:]
    atab = jnp.dot(xt, a_ref[...], preferred_element_type=jnp.float32)
    atab_ref[...] = atab
    mx = jnp.max(atab, axis=0, keepdims=True)
    msum = mx[:, 0:2] + mx[:, 2:4]
    msum = jnp.where(msum >= 0.0, msum, msum * 0.2)
    m_ref[...] = jnp.concatenate(
        [jnp.broadcast_to(msum[:, 0:1], (1, 16)),
         jnp.broadcast_to(msum[:, 1:2], (1, 16))], axis=0)


def _run_head(xpad, wt, amat):
    return pl.pallas_call(
        _head_body,
        out_shape=(
            jax.ShapeDtypeStruct((2 * NP, D), jnp.float32),
            jax.ShapeDtypeStruct((NP, 16), jnp.float32),
            jax.ShapeDtypeStruct((2, 16), jnp.float32),
        ),
    )(xpad, wt, amat)


def _make_edge_logits(epp, nch):
    epw = epp // 32

    @functools.partial(
        pl.kernel,
        out_type=(
            jax.ShapeDtypeStruct((2 * epp,), jnp.float32),
            jax.ShapeDtypeStruct((4 * NP,), jnp.float32),
        ),
        mesh=_mesh(),
        scratch_types=[
            pltpu.VMEM((C,), jnp.int32),
            pltpu.VMEM((C,), jnp.int32),
            pltpu.VMEM((C,), jnp.float32),
            pltpu.VMEM((C,), jnp.float32),
            pltpu.VMEM((C,), jnp.float32),
            pltpu.VMEM((C,), jnp.float32),
            pltpu.VMEM((C,), jnp.float32),
            pltpu.VMEM((C,), jnp.float32),
            pltpu.VMEM((2, 16), jnp.float32),
            pltpu.VMEM_SHARED((NP,), jnp.float32),
            pltpu.VMEM_SHARED((NP,), jnp.float32),
            pltpu.SemaphoreType.DMA,
            pltpu.SemaphoreType.DMA,
            pltpu.SemaphoreType.DMA,
            pltpu.SemaphoreType.DMA,
        ],
    )
    def edge_logits(src_hbm, dst_hbm, as0_hbm, as1_hbm, ad0_hbm, ad1_hbm,
                    m_hbm,
                    w_out, pden_out,
                    srcv, dstv, s0v, s1v, d0v, d1v, w0, w1, mv,
                    den0, den1, sem, sem2, sem3, sem4):
        c = lax.axis_index("c")
        s = lax.axis_index("s")
        wid = s * 2 + c
        base_n = s * SPAN
        pltpu.sync_copy(m_hbm, mv)
        for k in range(C // 16):
            w0[pl.ds(k * 16, 16)] = jnp.zeros((16,), jnp.float32)
        for off, ln in ((0, C), (C, C), (2 * C, C), (3 * C, C), (4 * C, 120)):
            pltpu.sync_copy(w0.at[pl.ds(0, ln)],
                            den0.at[pl.ds(base_n + off, ln)])
            pltpu.sync_copy(w0.at[pl.ds(0, ln)],
                            den1.at[pl.ds(base_n + off, ln)])
        plsc.subcore_barrier()
        m0 = mv[0]
        m1 = mv[1]

        def chunk(i, carry):
            base = wid * epw + i * C
            c1 = pltpu.async_copy(src_hbm.at[pl.ds(base, C)], srcv, sem)
            c2 = pltpu.async_copy(dst_hbm.at[pl.ds(base, C)], dstv, sem2)
            c1.wait()
            c2.wait()
            g1 = pltpu.async_copy(as0_hbm.at[srcv], s0v, sem)
            g2 = pltpu.async_copy(as1_hbm.at[srcv], s1v, sem2)
            g3 = pltpu.async_copy(ad0_hbm.at[dstv], d0v, sem3)
            g4 = pltpu.async_copy(ad1_hbm.at[dstv], d1v, sem4)
            g1.wait()
            g2.wait()
            g3.wait()
            g4.wait()
            for k in range(C // 16):
                q = pl.ds(k * 16, 16)
                al0 = s0v[q] + d0v[q]
                al0 = jnp.where(al0 >= 0.0, al0, al0 * 0.2)
                w0[q] = jnp.exp(al0 - m0)
                al1 = s1v[q] + d1v[q]
                al1 = jnp.where(al1 >= 0.0, al1, al1 * 0.2)
                w1[q] = jnp.exp(al1 - m1)
            o1 = pltpu.async_copy(w0, w_out.at[pl.ds(base, C)], sem)
            o2 = pltpu.async_copy(w1, w_out.at[pl.ds(epp + base, C)], sem2)
            o3 = pltpu.async_copy(w0, den0.at[dstv], sem3, add=True)
            o4 = pltpu.async_copy(w1, den1.at[dstv], sem4, add=True)
            o1.wait()
            o2.wait()
            o3.wait()
            o4.wait()
            return carry

        lax.fori_loop(0, nch, chunk, 0)
        plsc.subcore_barrier()
        for off, ln in ((0, C), (C, C), (2 * C, C), (3 * C, C), (4 * C, 120)):
            pltpu.sync_copy(den0.at[pl.ds(base_n + off, ln)],
                            w0.at[pl.ds(0, ln)])
            pltpu.sync_copy(
                w0.at[pl.ds(0, ln)],
                pden_out.at[pl.ds(c * 2 * NP + base_n + off, ln)])
            pltpu.sync_copy(den1.at[pl.ds(base_n + off, ln)],
                            w1.at[pl.ds(0, ln)])
            pltpu.sync_copy(
                w1.at[pl.ds(0, ln)],
                pden_out.at[pl.ds(c * 2 * NP + NP + base_n + off, ln)])

    return edge_logits


def _make_aggregate(epp, nch):
    epw = epp // 16

    @functools.partial(
        pl.kernel,
        out_type=jax.ShapeDtypeStruct((2 * NP, D), jnp.float32),
        mesh=_mesh(),
        scratch_types=[
            pltpu.VMEM((C,), jnp.int32),
            pltpu.VMEM((C,), jnp.int32),
            pltpu.VMEM((C,), jnp.int32),
            pltpu.VMEM((C,), jnp.int32),
            pltpu.VMEM((C,), jnp.int32),
            pltpu.VMEM((C,), jnp.float32),
            pltpu.VMEM((C,), jnp.float32),
            pltpu.VMEM((C,), jnp.float32),
            pltpu.VMEM((C, D), jnp.float32),
            pltpu.VMEM_SHARED((NP, D), jnp.float32),
            pltpu.SemaphoreType.DMA,
            pltpu.SemaphoreType.DMA,
            pltpu.SemaphoreType.DMA,
        ],
    )
    def aggregate(src_hbm, dst_hbm, w_hbm, pden_hbm, xt_hbm,
                  agg_out,
                  srcv, dstv, i1v, i2v, i3v, wv, g1v, g2v, rows, agg_sp,
                  sem, sem2, sem3):
        c = lax.axis_index("c")
        s = lax.axis_index("s")
        base_n = s * SPAN

        def zr(j, cr):
            for r in range(D // 16):
                rows[j, pl.ds(r * 16, 16)] = jnp.zeros((16,), jnp.float32)
            return cr

        lax.fori_loop(0, C, zr, 0)
        for off, ln in ((0, C), (C, C), (2 * C, C), (3 * C, C), (4 * C, 120)):
            pltpu.sync_copy(rows.at[pl.ds(0, ln)],
                            agg_sp.at[pl.ds(base_n + off, ln)])
        plsc.subcore_barrier()

        def chunk(i, carry):
            base = s * epw + i * C
            c1 = pltpu.async_copy(src_hbm.at[pl.ds(base, C)], srcv, sem)
            c2 = pltpu.async_copy(dst_hbm.at[pl.ds(base, C)], dstv, sem2)
            c3 = pltpu.async_copy(w_hbm.at[pl.ds(c * epp + base, C)], wv, sem3)
            c1.wait()
            c2.wait()

            off1 = c * NP
            off2 = 2 * NP + c * NP
            for k in range(C // 16):
                q = pl.ds(k * 16, 16)
                dq = dstv[q]
                i1v[q] = dq + off1
                i2v[q] = dq + off2
                i3v[q] = srcv[q] + off1
            g1 = pltpu.async_copy(pden_hbm.at[i1v], g1v, sem)
            g2 = pltpu.async_copy(pden_hbm.at[i2v], g2v, sem2)
            gx = pltpu.async_copy(xt_hbm.at[i3v], rows, sem3)
            c3.wait()
            g1.wait()
            g2.wait()
            gx.wait()

            for k in range(C // 16):
                q = pl.ds(k * 16, 16)
                cf = wv[q] / (g1v[q] + g2v[q] + 1e-16)
                for u in range(16):
                    e = k * 16 + u
                    cfu = cf.at[jnp.full((16,), u, jnp.int32)].get(
                        mode="promise_in_bounds")
                    for r in range(D // 16):
                        q2 = pl.ds(r * 16, 16)
                        rows[e, q2] = rows[e, q2] * cfu
            pltpu.sync_copy(rows, agg_sp.at[dstv], add=True)
            return carry

        lax.fori_loop(0, nch, chunk, 0)
        plsc.subcore_barrier()
        for off, ln in ((0, C), (C, C), (2 * C, C), (3 * C, C), (4 * C, 120)):
            pltpu.sync_copy(agg_sp.at[pl.ds(base_n + off, ln)],
                            rows.at[pl.ds(0, ln)])
            pltpu.sync_copy(rows.at[pl.ds(0, ln)],
                            agg_out.at[pl.ds(c * NP + base_n + off, ln)])

    return aggregate


def _tail_body(agg_ref, na_ref, bg_ref, wl_ref, l1w_ref, l1b_ref,
               w1_ref, b1_ref, w2_ref, b2_ref, l2w_ref, l2b_ref, out_ref):
    whole = agg_ref[...]
    gat = jnp.concatenate([whole[:N], whole[NP:NP + N]], axis=1) + bg_ref[...]
    x1 = jnp.dot(gat, wl_ref[...], preferred_element_type=jnp.float32)
    x1 = x1 + na_ref[...]
    mu = jnp.mean(x1)
    ce = x1 - mu
    var = jnp.mean(ce * ce)
    x = ce * lax.rsqrt(var + 1e-5) * l1w_ref[...] + l1b_ref[...]
    h = jnp.maximum(
        jnp.dot(x, w1_ref[...], preferred_element_type=jnp.float32)
        + b1_ref[...], 0.0)
    h2 = jnp.dot(h, w2_ref[...], preferred_element_type=jnp.float32)
    y = x + h2 + b2_ref[...]
    mu2 = jnp.mean(y)
    ce2 = y - mu2
    var2 = jnp.mean(ce2 * ce2)
    out_ref[...] = ce2 * lax.rsqrt(var2 + 1e-5) * l2w_ref[...] + l2b_ref[...]


def _run_tail(agg, node_attr, bg, wl, l1w, l1b, w1, b1, w2, b2, l2w, l2b):
    return pl.pallas_call(
        _tail_body,
        out_shape=jax.ShapeDtypeStruct((N, D), jnp.float32),
    )(agg, node_attr, bg, wl, l1w, l1b, w1, b1, w2, b2, l2w, l2b)


def kernel(node_attr, edge_index, W_gat, att_src, att_dst, b_gat, W_lin,
           ln1_w, ln1_b, W_ff1, b_ff1, W_ff2, b_ff2, ln2_w, ln2_b):
    e = edge_index.shape[1]
    ep = e + N
    epp = -(-ep // (32 * C)) * (32 * C)
    nch_a = epp // (32 * C)
    nch_b = epp // (16 * C)

    f32 = jnp.float32
    xpad = jnp.zeros((NP, D), f32).at[:N].set(node_attr)
    loop = jnp.arange(N, dtype=jnp.int32)
    padi = jnp.full((epp - ep,), N, jnp.int32)
    src = jnp.concatenate([edge_index[0].astype(jnp.int32), loop, padi])
    dst = jnp.concatenate([edge_index[1].astype(jnp.int32), loop, padi])

    wt = W_gat.T
    amat = jnp.zeros((2 * D, 16), f32)
    amat = amat.at[:D, 0].set(att_src[0, 0]).at[D:, 1].set(att_src[0, 1])
    amat = amat.at[:D, 2].set(att_dst[0, 0]).at[D:, 3].set(att_dst[0, 1])

    xt2, atab, m2 = _run_head(xpad, wt, amat)

    w_flat, pden = _make_edge_logits(epp, nch_a)(
        src, dst,
        atab[:, 0] + 0.0, atab[:, 1] + 0.0,
        atab[:, 2] + 0.0, atab[:, 3] + 0.0,
        m2)

    agg = _make_aggregate(epp, nch_b)(
        src, dst, w_flat, pden, xt2)

    return _run_tail(
        agg, node_attr, b_gat.reshape(1, 2 * D), W_lin.T,
        ln1_w.reshape(1, D), ln1_b.reshape(1, D),
        W_ff1.T, b_ff1.reshape(1, FF), W_ff2.T, b_ff2.reshape(1, D),
        ln2_w.reshape(1, D), ln2_b.reshape(1, D))
